# baseline (speedup 1.0000x reference)
import functools

import jax
import jax.numpy as jnp
from jax import lax
from jax.experimental import pallas as pl
from jax.experimental.pallas import tpu as pltpu
from jax.experimental.pallas import tpu_sc as plsc

_K0 = 928981903
_K1 = 3453687069
_ROT = (13, 15, 26, 6, 17, 29, 16, 24)

NROWS, NCOLS = 16384, 4096
SPLIT = 12288
NSC = NROWS - SPLIT
NW = 32
ROWS_PER_W = NSC // NW
CH = 8
VECS_PER_CH = CH * NCOLS // 16


def _threefry_bits(j):
    k0 = jnp.uint32(_K0)
    k1 = jnp.uint32(_K1)
    k2 = jnp.uint32(_K0 ^ _K1 ^ 0x1BD11BDA)
    ks = (k0, k1, k2)
    x0 = jnp.full_like(j, k0)
    x1 = j + k1
    for g in range(5):
        rots = _ROT[0:4] if g % 2 == 0 else _ROT[4:8]
        for r in rots:
            x0 = x0 + x1
            x1 = (x1 << jnp.uint32(r)) | (x1 >> jnp.uint32(32 - r))
            x1 = x1 ^ x0
        x0 = x0 + ks[(g + 1) % 3]
        x1 = x1 + ks[(g + 2) % 3] + jnp.uint32(g + 1)
    return x0 ^ x1


def _binarize(bits, x):
    m = (bits >> jnp.uint32(9)).astype(jnp.int32)
    u_scaled = m.astype(jnp.float32)
    thresh = (x + 1.0) * 4194304.0
    return jnp.where(u_scaled <= thresh, 1.0, -1.0).astype(jnp.float32)


def _tc_kernel(x_ref, y_ref, *, block_rows):
    i = pl.program_id(0)
    row = jax.lax.broadcasted_iota(jnp.uint32, x_ref.shape, 0)
    col = jax.lax.broadcasted_iota(jnp.uint32, x_ref.shape, 1)
    j = (jnp.uint32(i * block_rows) + row) * jnp.uint32(NCOLS) + col
    y_ref[...] = _binarize(_threefry_bits(j), x_ref[...])


def _sc_kernel(x_hbm, y_hbm, xbuf, ybuf):
    wid = lax.axis_index("s") * 2 + lax.axis_index("c")
    base_row = SPLIT + wid * ROWS_PER_W

    def chunk_body(ch, _):
        elem0 = (base_row + ch * CH) * NCOLS
        pltpu.sync_copy(x_hbm.at[pl.ds(elem0, CH * NCOLS)], xbuf)

        def vec_body(v, _):
            off = v * 16
            j = (jnp.uint32(elem0) + jnp.uint32(off).astype(jnp.uint32)
                 + lax.iota(jnp.uint32, 16))
            xv = xbuf[pl.ds(off, 16)]
            ybuf[pl.ds(off, 16)] = _binarize(_threefry_bits(j), xv)
            return 0

        lax.fori_loop(0, VECS_PER_CH, vec_body, 0)
        out0 = (base_row - SPLIT + ch * CH) * NCOLS
        pltpu.sync_copy(ybuf, y_hbm.at[pl.ds(out0, CH * NCOLS)])
        return 0

    lax.fori_loop(0, ROWS_PER_W // CH, chunk_body, 0)


@jax.jit
def kernel(x):
    block_rows = 256
    y_top = pl.pallas_call(
        functools.partial(_tc_kernel, block_rows=block_rows),
        grid=(SPLIT // block_rows,),
        in_specs=[pl.BlockSpec((block_rows, NCOLS), lambda i: (i, 0))],
        out_specs=pl.BlockSpec((block_rows, NCOLS), lambda i: (i, 0)),
        out_shape=jax.ShapeDtypeStruct((SPLIT, NCOLS), jnp.float32),
        compiler_params=pltpu.CompilerParams(
            dimension_semantics=("parallel",),
        ),
    )(x)

    mesh = plsc.VectorSubcoreMesh(
        core_axis_name="c", subcore_axis_name="s", num_cores=2, num_subcores=16
    )
    sc_call = functools.partial(
        pl.kernel,
        mesh=mesh,
        out_type=jax.ShapeDtypeStruct((NSC * NCOLS,), jnp.float32),
        scratch_types=[
            pltpu.VMEM((CH * NCOLS,), jnp.float32),
            pltpu.VMEM((CH * NCOLS,), jnp.float32),
        ],
    )(_sc_kernel)
    y_bot = sc_call(x.reshape(-1)).reshape(NSC, NCOLS)

    return jnp.concatenate([y_top, y_bot], axis=0)


# speedup vs baseline: 1.4314x; 1.4314x over previous
import functools

import jax
import jax.numpy as jnp
from jax import lax
from jax.experimental import pallas as pl
from jax.experimental.pallas import tpu as pltpu
from jax.experimental.pallas import tpu_sc as plsc

_K0 = 928981903
_K1 = 3453687069
_ROT = (13, 15, 26, 6, 17, 29, 16, 24)

NROWS, NCOLS = 16384, 4096
SPLIT = 12288
NSC = NROWS - SPLIT
NW = 32
ROWS_PER_W = NSC // NW
CH = 8


def _threefry_bits(j):
    k0 = jnp.uint32(_K0)
    k1 = jnp.uint32(_K1)
    k2 = jnp.uint32(_K0 ^ _K1 ^ 0x1BD11BDA)
    ks = (k0, k1, k2)
    x0 = jnp.full_like(j, k0)
    x1 = j + k1
    for g in range(5):
        rots = _ROT[0:4] if g % 2 == 0 else _ROT[4:8]
        for r in rots:
            x0 = x0 + x1
            x1 = (x1 << jnp.uint32(r)) | (x1 >> jnp.uint32(32 - r))
            x1 = x1 ^ x0
        x0 = x0 + ks[(g + 1) % 3]
        x1 = x1 + ks[(g + 2) % 3] + jnp.uint32(g + 1)
    return x0 ^ x1


def _binarize(bits, x):
    m = (bits >> jnp.uint32(9)).astype(jnp.int32)
    u_scaled = m.astype(jnp.float32)
    thresh = (x + 1.0) * 4194304.0
    return jnp.where(u_scaled <= thresh, 1.0, -1.0).astype(jnp.float32)


def _tc_kernel(x_ref, y_ref, *, block_rows):
    i = pl.program_id(0)
    row = jax.lax.broadcasted_iota(jnp.uint32, x_ref.shape, 0)
    col = jax.lax.broadcasted_iota(jnp.uint32, x_ref.shape, 1)
    j = (jnp.uint32(i * block_rows) + row) * jnp.uint32(NCOLS) + col
    y_ref[...] = _binarize(_threefry_bits(j), x_ref[...])


def _sc_kernel(x_hbm, y_hbm, xbuf, ybuf):
    wid = lax.axis_index("s") * 2 + lax.axis_index("c")
    base_row = SPLIT + wid * ROWS_PER_W

    def chunk_body(ch, _):
        row0 = base_row + ch * CH
        pltpu.sync_copy(x_hbm.at[pl.ds(row0, CH)], xbuf)
        for r in range(CH):
            def col_body(c, _, r=r):
                off = c * 16
                j = (jnp.uint32(row0 + r) * jnp.uint32(NCOLS)
                     + jnp.uint32(off) + lax.iota(jnp.uint32, 16))
                xv = xbuf[r, pl.ds(off, 16)]
                ybuf[r, pl.ds(off, 16)] = _binarize(_threefry_bits(j), xv)
                return 0

            lax.fori_loop(0, NCOLS // 16, col_body, 0)
        pltpu.sync_copy(ybuf, y_hbm.at[pl.ds(row0 - SPLIT, CH)])
        return 0

    lax.fori_loop(0, ROWS_PER_W // CH, chunk_body, 0)


def _assemble_kernel(yfull_ref, ybot_ref, out_ref):
    out_ref[...] = ybot_ref[...]


@jax.jit
def kernel(x):
    block_rows = 256

    y_full = pl.pallas_call(
        functools.partial(_tc_kernel, block_rows=block_rows),
        grid=(SPLIT // block_rows,),
        in_specs=[pl.BlockSpec((block_rows, NCOLS), lambda i: (i, 0))],
        out_specs=pl.BlockSpec((block_rows, NCOLS), lambda i: (i, 0)),
        out_shape=jax.ShapeDtypeStruct((NROWS, NCOLS), jnp.float32),
        compiler_params=pltpu.CompilerParams(
            dimension_semantics=("parallel",),
        ),
    )(x)

    mesh = plsc.VectorSubcoreMesh(
        core_axis_name="c", subcore_axis_name="s", num_cores=2, num_subcores=16
    )
    sc_call = functools.partial(
        pl.kernel,
        mesh=mesh,
        out_type=jax.ShapeDtypeStruct((NSC, NCOLS), jnp.float32),
        scratch_types=[
            pltpu.VMEM((CH, NCOLS), jnp.float32),
            pltpu.VMEM((CH, NCOLS), jnp.float32),
        ],
    )(_sc_kernel)
    y_bot = sc_call(x)

    nblk = NSC // block_rows
    off = SPLIT // block_rows
    return pl.pallas_call(
        _assemble_kernel,
        grid=(nblk,),
        in_specs=[
            pl.BlockSpec(memory_space=pl.ANY),
            pl.BlockSpec((block_rows, NCOLS), lambda i: (i, 0)),
        ],
        out_specs=pl.BlockSpec((block_rows, NCOLS), lambda i: (i + off, 0)),
        out_shape=jax.ShapeDtypeStruct((NROWS, NCOLS), jnp.float32),
        input_output_aliases={0: 0},
        compiler_params=pltpu.CompilerParams(
            dimension_semantics=("arbitrary",),
        ),
    )(y_full, y_bot)
